# baseline (speedup 1.0000x reference)
import jax, jax.numpy as jnp
import numpy as np
from jax.experimental import pallas as pl

N = 10000
E = 160000
D = 256
G = 64
NC = 2
GRID = 10
SIGMA = 1.0
K2 = GRID * GRID


def _mk_phi():
    gx = np.linspace(-1.0, 1.0, GRID)
    U = np.stack(np.meshgrid(gx, gx, indexing='ij'), axis=-1).reshape(-1, 2)
    d2 = ((U[:, None, :] - U[None, :, :]) ** 2).sum(-1)
    return jnp.asarray(np.exp(-d2 / (2.0 * SIGMA ** 2)), dtype=jnp.float32)


def _gc(x, src, dst, Wl, Wr, b):
    agg = jax.ops.segment_sum(x[src], dst, num_segments=N)
    return agg @ Wl.T + b + x @ Wr.T


def _bnorm(x, g, b, eps=1e-5):
    m = x.mean(axis=0)
    v = x.var(axis=0)
    return g * (x - m) / jnp.sqrt(v + eps) + b


def _resp(x, W, phi, beta=1.0):
    Y = phi @ W
    d2 = (x ** 2).sum(1, keepdims=True) + (Y ** 2).sum(1)[None, :] - 2.0 * (x @ Y.T)
    return jax.nn.softmax(-0.5 * beta * d2, axis=1)


def _pool3(h, batch):
    s = jax.ops.segment_sum(h, batch, num_segments=G)
    cnt = jax.ops.segment_sum(jnp.ones((h.shape[0], 1), h.dtype), batch, num_segments=G)
    avg = s / jnp.maximum(cnt, 1.0)
    mx = jax.ops.segment_max(h, batch, num_segments=G)
    return jnp.concatenate([avg, s, mx], axis=1)


def kernel(x, edge_index, batch, W1l, W1r, b1, W2l, W2r, b2, W3l, W3r, b3, WG, bG, Wg1, Wg2, Wg3, O1l, O1r, ob1, O2l, O2r, ob2, O3l, O3r, ob3, WO, bO, g1, g2, g3, og1, og2, og3, be1, be2, be3, obe1, obe2, obe3):
    src, dst = edge_index[0], edge_index[1]
    phi = _mk_phi()
    lr = jax.nn.leaky_relu
    x1 = _bnorm(lr(_gc(x, src, dst, W1l, W1r, b1)), g1, be1)
    x2 = _bnorm(lr(_gc(x1, src, dst, W2l, W2r, b2)), g2, be2)
    x3 = _bnorm(lr(_gc(x2, src, dst, W3l, W3r, b3)), g3, be3)
    h_conv = jnp.concatenate([x1, x2, x3], axis=1)
    h_GNN = _pool3(h_conv, batch)
    gnn_out = jax.nn.log_softmax(h_GNN @ WG.T + bG, axis=1)
    r1 = _resp(x1, Wg1, phi)
    r2 = _resp(x2, Wg2, phi)
    r3 = _resp(x3, Wg3, phi)
    r1 = jax.lax.stop_gradient(r1 / r1.max(axis=1, keepdims=True))
    r2 = jax.lax.stop_gradient(r2 / r2.max(axis=1, keepdims=True))
    r3 = jax.lax.stop_gradient(r3 / r3.max(axis=1, keepdims=True))
    h1 = _bnorm(lr(_gc(r1, src, dst, O1l, O1r, ob1)), og1, obe1)
    h2 = _bnorm(lr(_gc(r2, src, dst, O2l, O2r, ob2)), og2, obe2)
    h3 = _bnorm(lr(_gc(r3, src, dst, O3l, O3r, ob3)), og3, obe3)
    gtm_conv = jnp.concatenate([h1, h2, h3], axis=1)
    hp = _pool3(gtm_conv, batch)
    h = jax.nn.log_softmax(hp @ WO.T + bO, axis=1)
    return h, h_conv, gnn_out
